# baseline (speedup 1.0000x reference)
import dataclasses
import functools

import numpy as np
import jax
import jax.numpy as jnp
from jax import lax
from jax.experimental import pallas as pl
from jax.experimental.pallas import tpu as pltpu
from jax.experimental.pallas import tpu_sc as plsc

_NB = 3
_D = 256

_GUMBEL = np.array([0x3EAB0E4A, 0x3F73B78C, 0x3F39BC5F],
                   dtype=np.uint32).view(np.float32)


def kernel(X, W, b):
    g = _GUMBEL
    x = jnp.reshape(X, (_D,))
    w = jnp.reshape(W, (_NB * _D,))

    mesh = plsc.ScalarSubcoreMesh(axis_name="c", num_cores=1)
    cp = pltpu.CompilerParams()
    if "needs_layout_passes" in pltpu.CompilerParams.__dataclass_fields__:
        cp = dataclasses.replace(cp, needs_layout_passes=False)

    @functools.partial(
        pl.kernel,
        out_type=jax.ShapeDtypeStruct((1,), jnp.int32),
        mesh=mesh,
        compiler_params=cp,
        scratch_types=[
            pltpu.SMEM((_D,), jnp.float32),
            pltpu.SMEM((_D,), jnp.float32),
            pltpu.SMEM((_D,), jnp.float32),
            pltpu.SMEM((_D,), jnp.float32),
            pltpu.SMEM((_NB,), jnp.float32),
            pltpu.SMEM((1,), jnp.int32),
            pltpu.SemaphoreType.DMA,
        ],
    )
    def route(x_hbm, w_hbm, b_hbm, o_hbm, xs, w0, w1, w2, bs, os_, sem):
        cb = pltpu.async_copy(b_hbm, bs, sem)
        cx = pltpu.async_copy(x_hbm, xs, sem)
        c0 = pltpu.async_copy(w_hbm.at[pl.ds(0, _D)], w0, sem)
        c1 = pltpu.async_copy(w_hbm.at[pl.ds(_D, _D)], w1, sem)
        c2 = pltpu.async_copy(w_hbm.at[pl.ds(2 * _D, _D)], w2, sem)

        def dot_rows(ws):
            def body(j, acc):
                base = j * 4
                a = acc
                for k in range(4):
                    i = base + k
                    a = a + ws[i] * xs[i]
                return a
            return body

        cb.wait()
        cx.wait()
        c0.wait()
        s0 = lax.fori_loop(0, _D // 4, dot_rows(w0), bs[0] + float(g[0]))
        c1.wait()
        s1 = lax.fori_loop(0, _D // 4, dot_rows(w1), bs[1] + float(g[1]))
        c2.wait()
        s2 = lax.fori_loop(0, _D // 4, dot_rows(w2), bs[2] + float(g[2]))

        i01 = jnp.where(s1 > s0, 1, 0)
        best = jnp.maximum(s0, s1)
        os_[0] = jnp.where(s2 > best, 2, i01).astype(jnp.int32)
        pltpu.async_copy(os_, o_hbm, sem).wait()

    return route(x, w, b)


# speedup vs baseline: 1.0275x; 1.0275x over previous
import dataclasses
import functools

import numpy as np
import jax
import jax.numpy as jnp
from jax import lax
from jax.experimental import pallas as pl
from jax.experimental.pallas import tpu as pltpu
from jax.experimental.pallas import tpu_sc as plsc

_NB = 3
_D = 256

_GUMBEL = np.array([0x3EAB0E4A, 0x3F73B78C, 0x3F39BC5F],
                   dtype=np.uint32).view(np.float32)


def kernel(X, W, b):
    g = _GUMBEL
    x = jnp.reshape(X, (_D,))
    w = jnp.reshape(W, (_NB * _D,))

    mesh = plsc.ScalarSubcoreMesh(axis_name="c", num_cores=1)
    cp = pltpu.CompilerParams()
    if "needs_layout_passes" in pltpu.CompilerParams.__dataclass_fields__:
        cp = dataclasses.replace(cp, needs_layout_passes=False)

    @functools.partial(
        pl.kernel,
        out_type=jax.ShapeDtypeStruct((1,), jnp.int32),
        mesh=mesh,
        compiler_params=cp,
        scratch_types=[
            pltpu.SMEM((_D,), jnp.float32),
            pltpu.SMEM((_NB * _D,), jnp.float32),
            pltpu.SMEM((_NB,), jnp.float32),
            pltpu.SMEM((1,), jnp.int32),
            pltpu.SemaphoreType.DMA,
        ],
    )
    def route(x_hbm, w_hbm, b_hbm, o_hbm, xs, ws, bs, os_, sem):
        cx = pltpu.async_copy(x_hbm, xs, sem)
        cw = pltpu.async_copy(w_hbm, ws, sem)
        cb = pltpu.async_copy(b_hbm, bs, sem)
        cx.wait()
        cw.wait()
        cb.wait()

        def body(j, carry):
            a0, a1, a2 = carry
            base = j * 4
            for k in range(4):
                i = base + k
                xi = xs[i]
                a0 = a0 + ws[i] * xi
                a1 = a1 + ws[_D + i] * xi
                a2 = a2 + ws[2 * _D + i] * xi
            return a0, a1, a2

        init = (bs[0] + float(g[0]), bs[1] + float(g[1]), bs[2] + float(g[2]))
        s0, s1, s2 = lax.fori_loop(0, _D // 4, body, init)

        i01 = jnp.where(s1 > s0, 1, 0)
        best = jnp.maximum(s0, s1)
        os_[0] = jnp.where(s2 > best, 2, i01).astype(jnp.int32)
        pltpu.async_copy(os_, o_hbm, sem).wait()

    return route(x, w, b)
